# baseline (speedup 1.0000x reference)
import functools

import jax
import jax.numpy as jnp
from jax import lax
from jax.experimental import pallas as pl
from jax.experimental.pallas import tpu as pltpu
from jax.experimental.pallas import tpu_sc as plsc

N = 10000
E = 320000
D = 128
NP = 10240
NC = 2
NS = 16
NW = NC * NS
CCH = 64
EP = 327680
RW = EP // CCH
WR = RW // NW
KB = 16
NBLK = WR // KB
WR0 = 192
WR1 = 2 * WR - WR0
NRING = 4
PADV = N + 16
RPT = NP // NS
ZB = 32

_mesh = plsc.VectorSubcoreMesh(core_axis_name="c", subcore_axis_name="s")


def _zero_vmem_2d(ref, rows, cols):
    def body(i, carry):
        for k in range(cols // 16):
            ref[i, pl.ds(k * 16, 16)] = jnp.zeros((16,), jnp.float32)
        return carry
    lax.fori_loop(0, rows, body, 0)


@functools.partial(
    pl.kernel,
    out_type=(
        jax.ShapeDtypeStruct((NC, NP), jnp.float32),
        jax.ShapeDtypeStruct((NC, NP), jnp.float32),
    ),
    mesh=_mesh,
    scratch_types=[
        pltpu.VMEM((KB, CCH), jnp.int32),
        pltpu.VMEM((KB, CCH), jnp.int32),
        pltpu.VMEM((CCH,), jnp.float32),
        pltpu.VMEM_SHARED((NP,), jnp.float32),
        pltpu.VMEM_SHARED((NP,), jnp.float32),
        pltpu.SemaphoreType.DMA,
        pltpu.SemaphoreType.DMA,
        pltpu.SemaphoreType.DMA,
        pltpu.SemaphoreType.DMA,
    ],
)
def _sc_degrees(src_hbm, dst_hbm, ones_hbm, zcol_hbm, dego_out, degi_out,
                idxs_blk, idxd_blk, ones, dego_sh, degi_sh,
                so0, so1, si0, si1):
    c = lax.axis_index("c")
    s = lax.axis_index("s")
    w = s * NC + c
    sosem = [so0, so1]
    sisem = [si0, si1]

    pltpu.sync_copy(ones_hbm, ones)
    pltpu.sync_copy(zcol_hbm, dego_sh.at[pl.ds(s * RPT, RPT)])
    pltpu.sync_copy(zcol_hbm, degi_sh.at[pl.ds(s * RPT, RPT)])
    plsc.subcore_barrier()

    wbase = w * WR

    def blk_body(blk, carry):
        row0 = wbase + blk * KB
        pltpu.sync_copy(src_hbm.at[pl.ds(row0, KB)], idxs_blk)
        pltpu.sync_copy(dst_hbm.at[pl.ds(row0, KB)], idxd_blk)
        descs = []
        for i in range(KB):
            p = i % 2
            if i >= 2:
                descs[i - 2][0].wait()
                descs[i - 2][1].wait()
            do = pltpu.async_copy(ones, dego_sh.at[idxs_blk.at[i]],
                                  sosem[p], add=True)
            di = pltpu.async_copy(ones, degi_sh.at[idxd_blk.at[i]],
                                  sisem[p], add=True)
            descs.append((do, di))
        for i in (KB - 2, KB - 1):
            descs[i][0].wait()
            descs[i][1].wait()
        return carry
    lax.fori_loop(0, NBLK, blk_body, 0)
    plsc.subcore_barrier()

    pltpu.sync_copy(dego_sh.at[pl.ds(s * RPT, RPT)],
                    dego_out.at[c, pl.ds(s * RPT, RPT)])
    pltpu.sync_copy(degi_sh.at[pl.ds(s * RPT, RPT)],
                    degi_out.at[c, pl.ds(s * RPT, RPT)])


def _make_spmm(compute_u):
    out_type = [jax.ShapeDtypeStruct((NC, NP, D), jnp.float32)]
    scratch = [
        pltpu.VMEM((KB, CCH), jnp.int32),
        pltpu.VMEM((KB, CCH), jnp.int32),
    ] + [pltpu.VMEM((CCH, D), jnp.float32) for _ in range(NRING)] + [
        pltpu.VMEM((ZB, D), jnp.float32),
        pltpu.VMEM_SHARED((NP, D), jnp.float32),
    ] + [pltpu.SemaphoreType.DMA for _ in range(NRING)]
    if compute_u:
        out_type.append(jax.ShapeDtypeStruct((NC, NP), jnp.float32))
        scratch += (
            [pltpu.VMEM((CCH,), jnp.float32) for _ in range(NRING)]
            + [pltpu.VMEM_SHARED((NP,), jnp.float32)]
            + [pltpu.SemaphoreType.DMA for _ in range(NRING)]
        )

    def body(ha_hbm, hb_hbm, hc_hbm, hd_hbm, src_hbm, dst_hbm, nd_hbm,
             zcol_hbm, *rest):
        if compute_u:
            (agg_out, u_out, idxs_blk, idxd_blk, *rest2) = rest
            rows = rest2[:NRING]
            zbuf, agg_sh = rest2[NRING:NRING + 2]
            gsem = rest2[NRING + 2:2 * NRING + 2]
            vals = rest2[2 * NRING + 2:3 * NRING + 2]
            u_sh = rest2[3 * NRING + 2]
            vsem = rest2[3 * NRING + 3:]
        else:
            (agg_out, idxs_blk, idxd_blk, *rest2) = rest
            rows = rest2[:NRING]
            zbuf, agg_sh = rest2[NRING:NRING + 2]
            gsem = rest2[NRING + 2:]
        c = lax.axis_index("c")
        s = lax.axis_index("s")

        _zero_vmem_2d(zbuf, ZB, D)
        for t in range(RPT // ZB):
            pltpu.sync_copy(zbuf, agg_sh.at[pl.ds(s * RPT + t * ZB, ZB)])
        if compute_u:
            pltpu.sync_copy(zcol_hbm, u_sh.at[pl.ds(s * RPT, RPT)])
        plsc.subcore_barrier()

        wbase = jnp.where(c == 0, s * WR0, NS * WR0 + s * WR1)
        nblk = jnp.where(c == 0, WR0 // KB, WR1 // KB)

        def issue(i, g, v):
            p = i % NRING
            tbl = (ha_hbm, hb_hbm, hc_hbm, hd_hbm)[i % 4]
            g[i] = pltpu.async_copy(tbl.at[idxs_blk.at[i]], rows[p],
                                    gsem[p])
            if compute_u:
                v[i] = pltpu.async_copy(nd_hbm.at[idxd_blk.at[i]], vals[p],
                                        vsem[p])

        def blk_body(blk, carry):
            row0 = wbase + blk * KB
            pltpu.sync_copy(src_hbm.at[pl.ds(row0, KB)], idxs_blk)
            pltpu.sync_copy(dst_hbm.at[pl.ds(row0, KB)], idxd_blk)
            g = [None] * KB
            v = [None] * KB
            for pre in range(NRING - 1):
                issue(pre, g, v)
            for i in range(KB):
                p = i % NRING
                if i + NRING - 1 < KB:
                    issue(i + NRING - 1, g, v)
                g[i].wait()
                pltpu.sync_copy(rows[p], agg_sh.at[idxd_blk.at[i]], add=True)
                if compute_u:
                    v[i].wait()
                    pltpu.sync_copy(vals[p], u_sh.at[idxs_blk.at[i]],
                                    add=True)
            return carry
        lax.fori_loop(0, nblk, blk_body, 0)
        plsc.subcore_barrier()

        for t in range(RPT // 128):
            pltpu.sync_copy(agg_sh.at[pl.ds(s * RPT + t * 128, 128)],
                            agg_out.at[c, pl.ds(s * RPT + t * 128, 128)])
        if compute_u:
            pltpu.sync_copy(u_sh.at[pl.ds(s * RPT, RPT)],
                            u_out.at[c, pl.ds(s * RPT, RPT)])

    out_t = tuple(out_type) if compute_u else out_type[0]
    return pl.kernel(body, out_type=out_t, mesh=_mesh,
                     scratch_types=scratch)


_sc_spmm_u = _make_spmm(True)
_sc_spmm = _make_spmm(False)


_BR = 1280
_GRID = NP // _BR


def _tc_norms_h1_body(x0_ref, w1_ref, dego_ref, degi_ref,
                      *out_refs):
    h1_refs = out_refs[:4]
    ns_ref, nd_ref = out_refs[4:]
    do_ = dego_ref[0] + dego_ref[1]
    di = degi_ref[0] + degi_ref[1]
    ns = lax.rsqrt(jnp.maximum(do_, 1.0))
    nd = lax.rsqrt(jnp.maximum(di, 1.0))
    ns_ref[...] = ns
    nd_ref[...] = nd
    h = jnp.dot(x0_ref[...], w1_ref[...],
                preferred_element_type=jnp.float32, precision=lax.Precision.HIGHEST) * ns
    for r in h1_refs:
        r[...] = h


def _tc_norms_h1(x0, w1, dego, degi):
    return pl.pallas_call(
        _tc_norms_h1_body,
        grid=(_GRID,),
        in_specs=[
            pl.BlockSpec((_BR, D), lambda i: (i, 0)),
            pl.BlockSpec((D, D), lambda i: (0, 0)),
            pl.BlockSpec((NC, _BR, 1), lambda i: (0, i, 0)),
            pl.BlockSpec((NC, _BR, 1), lambda i: (0, i, 0)),
        ],
        out_specs=[pl.BlockSpec((_BR, D), lambda i: (i, 0))] * 4 + [
            pl.BlockSpec((_BR, 1), lambda i: (i, 0)),
            pl.BlockSpec((_BR, 1), lambda i: (i, 0)),
        ],
        out_shape=[jax.ShapeDtypeStruct((NP, D), jnp.float32)] * 4 + [
            jax.ShapeDtypeStruct((NP, 1), jnp.float32),
            jax.ShapeDtypeStruct((NP, 1), jnp.float32),
        ],
    )(x0, w1, dego, degi)


def _tc_layer_body(agg_ref, nd_ref, b_ref, w_next_ref, ns_ref, u_ref,
                   *out_refs):
    h_refs = out_refs[:4]
    wvec_ref = out_refs[4]
    i = pl.program_id(0)
    a = agg_ref[0] + agg_ref[1]
    x = jnp.maximum(a * nd_ref[...] + b_ref[...], 0.0)
    h = jnp.dot(x, w_next_ref[...],
                preferred_element_type=jnp.float32, precision=lax.Precision.HIGHEST) * ns_ref[...]
    for r in h_refs:
        r[...] = h
    row_ids = lax.broadcasted_iota(jnp.int32, (_BR, 1), 0) + i * _BR
    wv = ns_ref[...] * (u_ref[0] + u_ref[1])
    wvec_ref[...] = jnp.where(row_ids < N, wv, 0.0)


def _tc_layer(agg, nd, b, w_next, ns, u):
    return pl.pallas_call(
        _tc_layer_body,
        grid=(_GRID,),
        in_specs=[
            pl.BlockSpec((NC, _BR, D), lambda i: (0, i, 0)),
            pl.BlockSpec((_BR, 1), lambda i: (i, 0)),
            pl.BlockSpec((1, D), lambda i: (0, 0)),
            pl.BlockSpec((D, D), lambda i: (0, 0)),
            pl.BlockSpec((_BR, 1), lambda i: (i, 0)),
            pl.BlockSpec((NC, _BR, 1), lambda i: (0, i, 0)),
        ],
        out_specs=[pl.BlockSpec((_BR, D), lambda i: (i, 0))] * 4 + [
            pl.BlockSpec((_BR, 1), lambda i: (i, 0)),
        ],
        out_shape=[jax.ShapeDtypeStruct((NP, D), jnp.float32)] * 4 + [
            jax.ShapeDtypeStruct((NP, 1), jnp.float32),
        ],
    )(agg, nd, b, w_next, ns, u)


def _tc_reduce_body(agg_ref, nd_ref, b_ref, wvec_ref, r_ref):
    i = pl.program_id(0)
    a = agg_ref[0] + agg_ref[1]
    x2 = jnp.maximum(a * nd_ref[...] + b_ref[...], 0.0)
    partial = jnp.sum(x2 * wvec_ref[...], axis=0, keepdims=True)

    @pl.when(i == 0)
    def _():
        r_ref[...] = jnp.zeros_like(r_ref)
    r_ref[...] += partial


def _tc_reduce(agg, nd, b, wvec):
    return pl.pallas_call(
        _tc_reduce_body,
        grid=(_GRID,),
        in_specs=[
            pl.BlockSpec((NC, _BR, D), lambda i: (0, i, 0)),
            pl.BlockSpec((_BR, 1), lambda i: (i, 0)),
            pl.BlockSpec((1, D), lambda i: (0, 0)),
            pl.BlockSpec((_BR, 1), lambda i: (i, 0)),
        ],
        out_specs=pl.BlockSpec((1, D), lambda i: (0, 0)),
        out_shape=jax.ShapeDtypeStruct((1, D), jnp.float32),
    )(agg, nd, b, wvec)


def _tc_head_body(r_ref, w3_ref, b3_ref, fg_ref, lw1a_ref, lw1b_ref, lb1_ref,
                  lw2_ref, lb2_ref, lw3_ref, lb3_ref, out_ref):
    y = jnp.dot(r_ref[...], w3_ref[...],
                preferred_element_type=jnp.float32, precision=lax.Precision.HIGHEST) * (1.0 / N) + b3_ref[...]
    t = (jnp.dot(y, lw1a_ref[...], preferred_element_type=jnp.float32, precision=lax.Precision.HIGHEST)
         + jnp.dot(fg_ref[...], lw1b_ref[...],
                   preferred_element_type=jnp.float32, precision=lax.Precision.HIGHEST) + lb1_ref[...])
    t = jnp.maximum(t, 0.0)
    t = jnp.maximum(jnp.dot(t, lw2_ref[...],
                            preferred_element_type=jnp.float32, precision=lax.Precision.HIGHEST)
                    + lb2_ref[...], 0.0)
    out_ref[...] = jnp.dot(t, lw3_ref[...],
                           preferred_element_type=jnp.float32, precision=lax.Precision.HIGHEST) + lb3_ref[...]


def _tc_head(r, w3, b3, fg, lw1a, lw1b, lb1, lw2, lb2, lw3, lb3):
    return pl.pallas_call(
        _tc_head_body,
        out_shape=jax.ShapeDtypeStruct((1, 1), jnp.float32),
    )(r, w3, b3, fg, lw1a, lw1b, lb1, lw2, lb2, lw3, lb3)


def kernel(feats_node, edge_index, feats_graph, W1, b1, W2, b2, W3, b3,
           lw1, lb1, lw2, lb2, lw3, lb3):
    src = jnp.pad(edge_index[0].astype(jnp.int32), (0, EP - E),
                  constant_values=PADV).reshape(RW, CCH)
    dst = jnp.pad(edge_index[1].astype(jnp.int32), (0, EP - E),
                  constant_values=PADV).reshape(RW, CCH)
    x0 = jnp.pad(feats_node, ((0, NP - N), (0, 0)))
    ones_col = jnp.ones((CCH,), jnp.float32)
    zcol = jnp.zeros((RPT,), jnp.float32)

    dego, degi = _sc_degrees(src, dst, ones_col, zcol)
    h1a, h1b, h1c, h1d, ns, nd = _tc_norms_h1(
        x0, W1, dego.reshape(NC, NP, 1), degi.reshape(NC, NP, 1))
    nd_flat = nd.reshape(NP)
    agg1, u = _sc_spmm_u(h1a, h1b, h1c, h1d, src, dst, nd_flat, zcol)
    h2a, h2b, h2c, h2d, wvec = _tc_layer(agg1, nd, b1.reshape(1, D), W2, ns,
                                         u.reshape(NC, NP, 1))
    agg2 = _sc_spmm(h2a, h2b, h2c, h2d, src, dst, nd_flat, zcol)
    r = _tc_reduce(agg2, nd, b2.reshape(1, D), wvec)

    fg = jnp.pad(feats_graph, ((0, 0), (0, 5)))
    lw1a = lw1[:D]
    lw1b = jnp.pad(lw1[D:], ((0, 5), (0, 0)))
    out = _tc_head(r, W3, b3.reshape(1, D), fg, lw1a, lw1b,
                   lb1.reshape(1, -1), lw2, lb2.reshape(1, -1),
                   lw3, lb3.reshape(1, 1))
    return out.reshape(-1)


# speedup vs baseline: 1.0452x; 1.0452x over previous
import functools

import jax
import jax.numpy as jnp
from jax import lax
from jax.experimental import pallas as pl
from jax.experimental.pallas import tpu as pltpu
from jax.experimental.pallas import tpu_sc as plsc

N = 10000
E = 320000
D = 128
NP = 10240
NC = 2
NS = 16
NW = NC * NS
CCH = 64
EP = 327680
RW = EP // CCH
WR = RW // NW
KB = 16
NBLK = WR // KB
WR0 = 208
WR1 = 2 * WR - WR0
NRING = 4
PADV = N + 16
RPT = NP // NS
ZB = 32

_mesh = plsc.VectorSubcoreMesh(core_axis_name="c", subcore_axis_name="s")


def _zero_vmem_2d(ref, rows, cols):
    def body(i, carry):
        for k in range(cols // 16):
            ref[i, pl.ds(k * 16, 16)] = jnp.zeros((16,), jnp.float32)
        return carry
    lax.fori_loop(0, rows, body, 0)


@functools.partial(
    pl.kernel,
    out_type=(
        jax.ShapeDtypeStruct((NC, NP), jnp.float32),
        jax.ShapeDtypeStruct((NC, NP), jnp.float32),
    ),
    mesh=_mesh,
    scratch_types=[
        pltpu.VMEM((KB, CCH), jnp.int32),
        pltpu.VMEM((KB, CCH), jnp.int32),
        pltpu.VMEM((CCH,), jnp.float32),
        pltpu.VMEM_SHARED((NP,), jnp.float32),
        pltpu.VMEM_SHARED((NP,), jnp.float32),
        pltpu.SemaphoreType.DMA,
        pltpu.SemaphoreType.DMA,
        pltpu.SemaphoreType.DMA,
        pltpu.SemaphoreType.DMA,
    ],
)
def _sc_degrees(src_hbm, dst_hbm, ones_hbm, zcol_hbm, dego_out, degi_out,
                idxs_blk, idxd_blk, ones, dego_sh, degi_sh,
                so0, so1, si0, si1):
    c = lax.axis_index("c")
    s = lax.axis_index("s")
    w = s * NC + c
    sosem = [so0, so1]
    sisem = [si0, si1]

    pltpu.sync_copy(ones_hbm, ones)
    pltpu.sync_copy(zcol_hbm, dego_sh.at[pl.ds(s * RPT, RPT)])
    pltpu.sync_copy(zcol_hbm, degi_sh.at[pl.ds(s * RPT, RPT)])
    plsc.subcore_barrier()

    wbase = w * WR

    def blk_body(blk, carry):
        row0 = wbase + blk * KB
        pltpu.sync_copy(src_hbm.at[pl.ds(row0, KB)], idxs_blk)
        pltpu.sync_copy(dst_hbm.at[pl.ds(row0, KB)], idxd_blk)
        descs = []
        for i in range(KB):
            p = i % 2
            if i >= 2:
                descs[i - 2][0].wait()
                descs[i - 2][1].wait()
            do = pltpu.async_copy(ones, dego_sh.at[idxs_blk.at[i]],
                                  sosem[p], add=True)
            di = pltpu.async_copy(ones, degi_sh.at[idxd_blk.at[i]],
                                  sisem[p], add=True)
            descs.append((do, di))
        for i in (KB - 2, KB - 1):
            descs[i][0].wait()
            descs[i][1].wait()
        return carry
    lax.fori_loop(0, NBLK, blk_body, 0)
    plsc.subcore_barrier()

    pltpu.sync_copy(dego_sh.at[pl.ds(s * RPT, RPT)],
                    dego_out.at[c, pl.ds(s * RPT, RPT)])
    pltpu.sync_copy(degi_sh.at[pl.ds(s * RPT, RPT)],
                    degi_out.at[c, pl.ds(s * RPT, RPT)])


def _make_spmm(compute_u):
    out_type = [jax.ShapeDtypeStruct((NC, NP, D), jnp.float32)]
    scratch = [
        pltpu.VMEM((KB, CCH), jnp.int32),
        pltpu.VMEM((KB, CCH), jnp.int32),
    ] + [pltpu.VMEM((CCH, D), jnp.float32) for _ in range(NRING)] + [
        pltpu.VMEM((ZB, D), jnp.float32),
        pltpu.VMEM_SHARED((NP, D), jnp.float32),
    ] + [pltpu.SemaphoreType.DMA for _ in range(NRING)]
    if compute_u:
        out_type.append(jax.ShapeDtypeStruct((NC, NP), jnp.float32))
        scratch += (
            [pltpu.VMEM((CCH,), jnp.float32) for _ in range(NRING)]
            + [pltpu.VMEM_SHARED((NP,), jnp.float32)]
            + [pltpu.SemaphoreType.DMA for _ in range(NRING)]
        )

    def body(ha_hbm, hb_hbm, hc_hbm, hd_hbm, src_hbm, dst_hbm, nd_hbm,
             zcol_hbm, *rest):
        if compute_u:
            (agg_out, u_out, idxs_blk, idxd_blk, *rest2) = rest
            rows = rest2[:NRING]
            zbuf, agg_sh = rest2[NRING:NRING + 2]
            gsem = rest2[NRING + 2:2 * NRING + 2]
            vals = rest2[2 * NRING + 2:3 * NRING + 2]
            u_sh = rest2[3 * NRING + 2]
            vsem = rest2[3 * NRING + 3:]
        else:
            (agg_out, idxs_blk, idxd_blk, *rest2) = rest
            rows = rest2[:NRING]
            zbuf, agg_sh = rest2[NRING:NRING + 2]
            gsem = rest2[NRING + 2:]
        c = lax.axis_index("c")
        s = lax.axis_index("s")

        _zero_vmem_2d(zbuf, ZB, D)
        for t in range(RPT // ZB):
            pltpu.sync_copy(zbuf, agg_sh.at[pl.ds(s * RPT + t * ZB, ZB)])
        if compute_u:
            pltpu.sync_copy(zcol_hbm, u_sh.at[pl.ds(s * RPT, RPT)])
        plsc.subcore_barrier()

        wbase = jnp.where(c == 0, s * WR0, NS * WR0 + s * WR1)
        nblk = jnp.where(c == 0, WR0 // KB, WR1 // KB)

        def issue(i, g, v):
            p = i % NRING
            tbl = (ha_hbm, hb_hbm, hc_hbm, hd_hbm)[i % 4]
            g[i] = pltpu.async_copy(tbl.at[idxs_blk.at[i]], rows[p],
                                    gsem[p])
            if compute_u:
                v[i] = pltpu.async_copy(nd_hbm.at[idxd_blk.at[i]], vals[p],
                                        vsem[p])

        def blk_body(blk, carry):
            row0 = wbase + blk * KB
            pltpu.sync_copy(src_hbm.at[pl.ds(row0, KB)], idxs_blk)
            pltpu.sync_copy(dst_hbm.at[pl.ds(row0, KB)], idxd_blk)
            g = [None] * KB
            v = [None] * KB
            for pre in range(NRING - 1):
                issue(pre, g, v)
            for i in range(KB):
                p = i % NRING
                if i + NRING - 1 < KB:
                    issue(i + NRING - 1, g, v)
                g[i].wait()
                pltpu.sync_copy(rows[p], agg_sh.at[idxd_blk.at[i]], add=True)
                if compute_u:
                    v[i].wait()
                    pltpu.sync_copy(vals[p], u_sh.at[idxs_blk.at[i]],
                                    add=True)
            return carry
        lax.fori_loop(0, nblk, blk_body, 0)
        plsc.subcore_barrier()

        for t in range(RPT // 128):
            pltpu.sync_copy(agg_sh.at[pl.ds(s * RPT + t * 128, 128)],
                            agg_out.at[c, pl.ds(s * RPT + t * 128, 128)])
        if compute_u:
            pltpu.sync_copy(u_sh.at[pl.ds(s * RPT, RPT)],
                            u_out.at[c, pl.ds(s * RPT, RPT)])

    out_t = tuple(out_type) if compute_u else out_type[0]
    return pl.kernel(body, out_type=out_t, mesh=_mesh,
                     scratch_types=scratch)


_sc_spmm_u = _make_spmm(True)
_sc_spmm = _make_spmm(False)


_BR = 1280
_GRID = NP // _BR


def _tc_norms_h1_body(x0_ref, w1_ref, dego_ref, degi_ref,
                      *out_refs):
    h1_refs = out_refs[:4]
    ns_ref, nd_ref = out_refs[4:]
    do_ = dego_ref[0] + dego_ref[1]
    di = degi_ref[0] + degi_ref[1]
    ns = lax.rsqrt(jnp.maximum(do_, 1.0))
    nd = lax.rsqrt(jnp.maximum(di, 1.0))
    ns_ref[...] = ns
    nd_ref[...] = nd
    h = jnp.dot(x0_ref[...], w1_ref[...],
                preferred_element_type=jnp.float32, precision=lax.Precision.HIGHEST) * ns
    for r in h1_refs:
        r[...] = h


def _tc_norms_h1(x0, w1, dego, degi):
    return pl.pallas_call(
        _tc_norms_h1_body,
        grid=(_GRID,),
        in_specs=[
            pl.BlockSpec((_BR, D), lambda i: (i, 0)),
            pl.BlockSpec((D, D), lambda i: (0, 0)),
            pl.BlockSpec((NC, _BR, 1), lambda i: (0, i, 0)),
            pl.BlockSpec((NC, _BR, 1), lambda i: (0, i, 0)),
        ],
        out_specs=[pl.BlockSpec((_BR, D), lambda i: (i, 0))] * 4 + [
            pl.BlockSpec((_BR, 1), lambda i: (i, 0)),
            pl.BlockSpec((_BR, 1), lambda i: (i, 0)),
        ],
        out_shape=[jax.ShapeDtypeStruct((NP, D), jnp.float32)] * 4 + [
            jax.ShapeDtypeStruct((NP, 1), jnp.float32),
            jax.ShapeDtypeStruct((NP, 1), jnp.float32),
        ],
    )(x0, w1, dego, degi)


def _tc_layer_body(agg_ref, nd_ref, b_ref, w_next_ref, ns_ref, u_ref,
                   *out_refs):
    h_refs = out_refs[:4]
    wvec_ref = out_refs[4]
    i = pl.program_id(0)
    a = agg_ref[0] + agg_ref[1]
    x = jnp.maximum(a * nd_ref[...] + b_ref[...], 0.0)
    h = jnp.dot(x, w_next_ref[...],
                preferred_element_type=jnp.float32, precision=lax.Precision.HIGHEST) * ns_ref[...]
    for r in h_refs:
        r[...] = h
    row_ids = lax.broadcasted_iota(jnp.int32, (_BR, 1), 0) + i * _BR
    wv = ns_ref[...] * (u_ref[0] + u_ref[1])
    wvec_ref[...] = jnp.where(row_ids < N, wv, 0.0)


def _tc_layer(agg, nd, b, w_next, ns, u):
    return pl.pallas_call(
        _tc_layer_body,
        grid=(_GRID,),
        in_specs=[
            pl.BlockSpec((NC, _BR, D), lambda i: (0, i, 0)),
            pl.BlockSpec((_BR, 1), lambda i: (i, 0)),
            pl.BlockSpec((1, D), lambda i: (0, 0)),
            pl.BlockSpec((D, D), lambda i: (0, 0)),
            pl.BlockSpec((_BR, 1), lambda i: (i, 0)),
            pl.BlockSpec((NC, _BR, 1), lambda i: (0, i, 0)),
        ],
        out_specs=[pl.BlockSpec((_BR, D), lambda i: (i, 0))] * 4 + [
            pl.BlockSpec((_BR, 1), lambda i: (i, 0)),
        ],
        out_shape=[jax.ShapeDtypeStruct((NP, D), jnp.float32)] * 4 + [
            jax.ShapeDtypeStruct((NP, 1), jnp.float32),
        ],
    )(agg, nd, b, w_next, ns, u)


def _tc_reduce_body(agg_ref, nd_ref, b_ref, wvec_ref, r_ref):
    i = pl.program_id(0)
    a = agg_ref[0] + agg_ref[1]
    x2 = jnp.maximum(a * nd_ref[...] + b_ref[...], 0.0)
    partial = jnp.sum(x2 * wvec_ref[...], axis=0, keepdims=True)

    @pl.when(i == 0)
    def _():
        r_ref[...] = jnp.zeros_like(r_ref)
    r_ref[...] += partial


def _tc_reduce(agg, nd, b, wvec):
    return pl.pallas_call(
        _tc_reduce_body,
        grid=(_GRID,),
        in_specs=[
            pl.BlockSpec((NC, _BR, D), lambda i: (0, i, 0)),
            pl.BlockSpec((_BR, 1), lambda i: (i, 0)),
            pl.BlockSpec((1, D), lambda i: (0, 0)),
            pl.BlockSpec((_BR, 1), lambda i: (i, 0)),
        ],
        out_specs=pl.BlockSpec((1, D), lambda i: (0, 0)),
        out_shape=jax.ShapeDtypeStruct((1, D), jnp.float32),
    )(agg, nd, b, wvec)


def _tc_head_body(r_ref, w3_ref, b3_ref, fg_ref, lw1a_ref, lw1b_ref, lb1_ref,
                  lw2_ref, lb2_ref, lw3_ref, lb3_ref, out_ref):
    y = jnp.dot(r_ref[...], w3_ref[...],
                preferred_element_type=jnp.float32, precision=lax.Precision.HIGHEST) * (1.0 / N) + b3_ref[...]
    t = (jnp.dot(y, lw1a_ref[...], preferred_element_type=jnp.float32, precision=lax.Precision.HIGHEST)
         + jnp.dot(fg_ref[...], lw1b_ref[...],
                   preferred_element_type=jnp.float32, precision=lax.Precision.HIGHEST) + lb1_ref[...])
    t = jnp.maximum(t, 0.0)
    t = jnp.maximum(jnp.dot(t, lw2_ref[...],
                            preferred_element_type=jnp.float32, precision=lax.Precision.HIGHEST)
                    + lb2_ref[...], 0.0)
    out_ref[...] = jnp.dot(t, lw3_ref[...],
                           preferred_element_type=jnp.float32, precision=lax.Precision.HIGHEST) + lb3_ref[...]


def _tc_head(r, w3, b3, fg, lw1a, lw1b, lb1, lw2, lb2, lw3, lb3):
    return pl.pallas_call(
        _tc_head_body,
        out_shape=jax.ShapeDtypeStruct((1, 1), jnp.float32),
    )(r, w3, b3, fg, lw1a, lw1b, lb1, lw2, lb2, lw3, lb3)


def kernel(feats_node, edge_index, feats_graph, W1, b1, W2, b2, W3, b3,
           lw1, lb1, lw2, lb2, lw3, lb3):
    src = jnp.pad(edge_index[0].astype(jnp.int32), (0, EP - E),
                  constant_values=PADV).reshape(RW, CCH)
    dst = jnp.pad(edge_index[1].astype(jnp.int32), (0, EP - E),
                  constant_values=PADV).reshape(RW, CCH)
    x0 = jnp.pad(feats_node, ((0, NP - N), (0, 0)))
    ones_col = jnp.ones((CCH,), jnp.float32)
    zcol = jnp.zeros((RPT,), jnp.float32)

    dego, degi = _sc_degrees(src, dst, ones_col, zcol)
    h1a, h1b, h1c, h1d, ns, nd = _tc_norms_h1(
        x0, W1, dego.reshape(NC, NP, 1), degi.reshape(NC, NP, 1))
    nd_flat = nd.reshape(NP)
    agg1, u = _sc_spmm_u(h1a, h1b, h1c, h1d, src, dst, nd_flat, zcol)
    h2a, h2b, h2c, h2d, wvec = _tc_layer(agg1, nd, b1.reshape(1, D), W2, ns,
                                         u.reshape(NC, NP, 1))
    agg2 = _sc_spmm(h2a, h2b, h2c, h2d, src, dst, nd_flat, zcol)
    r = _tc_reduce(agg2, nd, b2.reshape(1, D), wvec)

    fg = jnp.pad(feats_graph, ((0, 0), (0, 5)))
    lw1a = lw1[:D]
    lw1b = jnp.pad(lw1[D:], ((0, 5), (0, 0)))
    out = _tc_head(r, W3, b3.reshape(1, D), fg, lw1a, lw1b,
                   lb1.reshape(1, -1), lw2, lb2.reshape(1, -1),
                   lw3, lb3.reshape(1, 1))
    return out.reshape(-1)
